# Initial kernel scaffold; baseline (speedup 1.0000x reference)
import jax
import jax.numpy as jnp
from jax.experimental import pallas as pl


def kernel(x1, edge_index1, batch1, x2, edge_index2, Wl1, bl1, Wr1, Wl2, bl2, Wr2, fc1_w, fc1_b, g1, be1, fc2_w, fc2_b, g2, be2, fc3_w, fc3_b):
    raise NotImplementedError("write your pallas kernel here")


# speedup vs baseline: 2.2317x; 2.2317x over previous
import functools

import jax
import jax.numpy as jnp
from jax import lax
from jax.experimental import pallas as pl
from jax.experimental.pallas import tpu as pltpu
from jax.experimental.pallas import tpu_sc as plsc

_N1 = 10000
_E1 = 640000
_N2 = 199
_F = 128
_B = 16
_K = 50

_N1P = 10240

_NC = 2
_NS = 16
_NW = _NC * _NS
_EPW = _E1 // _NW
_CH = 80
_NCH = _EPW // _CH
_RPT = _N1P // _NS

_N2P = 256
_E2P = 3200

_NEG = float("-inf")


_NR = 3
_RNG = 3584
_NACC = _NR * _RNG
_RRT = _RNG // _NS
_SENT = -1


def _sc_mesh():
  return plsc.VectorSubcoreMesh(
      core_axis_name="c", subcore_axis_name="s",
      num_cores=_NC, num_subcores=_NS)


def _make_seg_kernel():
  scratch = [
      pltpu.VMEM((_CH,), jnp.int32),
      pltpu.VMEM((_CH,), jnp.int32),
      pltpu.VMEM((_CH,), jnp.int32),
      pltpu.VMEM((_CH,), jnp.int32),
      pltpu.VMEM((_CH, _F), jnp.float32),
      pltpu.SemaphoreType.DMA,
      pltpu.VMEM((_RRT, _F), jnp.float32),
      pltpu.VMEM_SHARED((_RNG, _F), jnp.float32),
  ]

  def body(x_hbm, src_hbm, dst_hbm, zf_hbm, msum_hbm,
           src_v, dst_v, sadj_v, dadj_v, rows_v, sem, bounce, accf):
    c = lax.axis_index("c")
    s = lax.axis_index("s")
    w = s * _NC + c

    pltpu.sync_copy(zf_hbm, bounce)
    for p in range(_NR):
      lo = p * _RNG
      hi = lo + _RNG
      pltpu.sync_copy(bounce, accf.at[pl.ds(s * _RRT, _RRT)])
      plsc.subcore_barrier()

      @pl.loop(0, _NCH)
      def _step(i):
        base = w * _EPW + i * _CH
        pltpu.sync_copy(src_hbm.at[pl.ds(base, _CH)], src_v)
        pltpu.sync_copy(dst_hbm.at[pl.ds(base, _CH)], dst_v)

        @pl.loop(0, _CH // 16)
        def _adj(j):
          sl = pl.ds(j * 16, 16)
          dv = dst_v[sl]
          sv = src_v[sl]
          m = (dv >= lo) & (dv < hi)
          sadj_v[sl] = jnp.where(m, sv, _SENT)
          dadj_v[sl] = jnp.where(m, dv - lo, _SENT)

        pltpu.async_copy(
            x_hbm.at[plsc.Indices(sadj_v, ignored_value=_SENT)],
            rows_v, sem).wait()
        pltpu.sync_copy(
            rows_v,
            accf.at[plsc.Indices(dadj_v, ignored_value=_SENT)], add=True)

      plsc.subcore_barrier()
      pltpu.sync_copy(accf.at[pl.ds(s * _RRT, _RRT)], bounce)
      pltpu.sync_copy(bounce,
                      msum_hbm.at[pl.ds(c * _NACC + lo + s * _RRT, _RRT)])
      plsc.subcore_barrier()
      if p + 1 < _NR:
        pltpu.sync_copy(zf_hbm, bounce)

  return pl.kernel(
      body,
      out_type=jax.ShapeDtypeStruct((_NC * _NACC, _F), jnp.float32),
      mesh=_sc_mesh(),
      scratch_types=scratch,
  )


def _make_cnt_kernel():
  scratch = [
      pltpu.VMEM((_CH,), jnp.int32),
      pltpu.VMEM((_CH,), jnp.int32),
      pltpu.VMEM((_CH, _F), jnp.float32),
      pltpu.VMEM((_RRT, _F), jnp.float32),
      pltpu.VMEM_SHARED((_RNG, _F), jnp.float32),
  ]

  def body(dst_hbm, zf_hbm, ones_hbm, cnt_hbm,
           dst_v, dadj_v, ones_v, bounce, accc):
    c = lax.axis_index("c")
    s = lax.axis_index("s")
    w = s * _NC + c

    pltpu.sync_copy(ones_hbm, ones_v)
    pltpu.sync_copy(zf_hbm, bounce)
    for p in range(_NR):
      lo = p * _RNG
      hi = lo + _RNG
      pltpu.sync_copy(bounce, accc.at[pl.ds(s * _RRT, _RRT)])
      plsc.subcore_barrier()

      @pl.loop(0, _NCH)
      def _step(i):
        base = w * _EPW + i * _CH
        pltpu.sync_copy(dst_hbm.at[pl.ds(base, _CH)], dst_v)

        @pl.loop(0, _CH // 16)
        def _adj(j):
          sl = pl.ds(j * 16, 16)
          dv = dst_v[sl]
          m = (dv >= lo) & (dv < hi)
          dadj_v[sl] = jnp.where(m, dv - lo, _SENT)

        pltpu.sync_copy(
            ones_v,
            accc.at[plsc.Indices(dadj_v, ignored_value=_SENT)], add=True)

      plsc.subcore_barrier()
      pltpu.sync_copy(accc.at[pl.ds(s * _RRT, _RRT)], bounce)
      pltpu.sync_copy(bounce,
                      cnt_hbm.at[pl.ds(c * _NACC + lo + s * _RRT, _RRT)])
      plsc.subcore_barrier()
      if p + 1 < _NR:
        pltpu.sync_copy(zf_hbm, bounce)

  return pl.kernel(
      body,
      out_type=jax.ShapeDtypeStruct((_NC * _NACC, _F), jnp.float32),
      mesh=_sc_mesh(),
      scratch_types=scratch,
  )


_seg_cache = {}


def _seg_sum(*args):
  if "seg" not in _seg_cache:
    _seg_cache["seg"] = _make_seg_kernel()
  return _seg_cache["seg"](*args)


def _seg_cnt(*args):
  if "cnt" not in _seg_cache:
    _seg_cache["cnt"] = _make_cnt_kernel()
  return _seg_cache["cnt"](*args)


def _tc_layer(msum_p, cnt_p, x, Wl, bl, Wr):
  n, f = x.shape
  fo = Wl.shape[1]
  blk = 2000
  grid = n // blk

  def body(ms_ref, cp_ref, x_ref, wl_ref, bl_ref, wr_ref, o_ref):
    ms = ms_ref[0] + ms_ref[1]
    cnt = cp_ref[0][:, 0:1] + cp_ref[1][:, 0:1]
    mean = ms / jnp.maximum(cnt, 1.0)
    h = (jnp.dot(mean, wl_ref[...], preferred_element_type=jnp.float32)
         + bl_ref[...]
         + jnp.dot(x_ref[...], wr_ref[...], preferred_element_type=jnp.float32))
    o_ref[...] = jnp.maximum(h, 0.0)

  return pl.pallas_call(
      body,
      grid=(grid,),
      in_specs=[
          pl.BlockSpec((2, blk, f), lambda i: (0, i, 0)),
          pl.BlockSpec((2, blk, f), lambda i: (0, i, 0)),
          pl.BlockSpec((blk, f), lambda i: (i, 0)),
          pl.BlockSpec((f, fo), lambda i: (0, 0)),
          pl.BlockSpec((1, fo), lambda i: (0, 0)),
          pl.BlockSpec((f, fo), lambda i: (0, 0)),
      ],
      out_specs=pl.BlockSpec((blk, fo), lambda i: (i, 0)),
      out_shape=jax.ShapeDtypeStruct((n, fo), jnp.float32),
  )(msum_p, cnt_p, x, Wl, bl, Wr)


def _tc_graph2(x2p, ei2p, Wl1, bl1, Wr1, Wl2, bl2, Wr2):
  def body(x_ref, ei_ref, wl1_ref, bl1_ref, wr1_ref, wl2_ref, bl2_ref,
           wr2_ref, o_ref):
    src = ei_ref[0:1, :]
    dst = ei_ref[1:2, :]
    ionn = lax.broadcasted_iota(jnp.int32, (_N2P, _E2P), 0)
    dm = (dst == ionn).astype(jnp.float32)
    sm = (src == ionn).astype(jnp.float32)
    a = lax.dot_general(dm, sm, (((1,), (1,)), ((), ())),
                        preferred_element_type=jnp.float32,
                        precision=lax.Precision.HIGHEST)
    cnt2 = jnp.maximum(jnp.sum(a, axis=1, keepdims=True), 1.0)
    x2 = x_ref[...]
    ms1 = jnp.dot(a, x2, preferred_element_type=jnp.float32,
                  precision=lax.Precision.HIGHEST)
    h = jnp.maximum(
        jnp.dot(ms1 / cnt2, wl1_ref[...], preferred_element_type=jnp.float32)
        + bl1_ref[...]
        + jnp.dot(x2, wr1_ref[...], preferred_element_type=jnp.float32), 0.0)
    ms2 = jnp.dot(a, h, preferred_element_type=jnp.float32,
                  precision=lax.Precision.HIGHEST)
    o_ref[...] = jnp.maximum(
        jnp.dot(ms2 / cnt2, wl2_ref[...], preferred_element_type=jnp.float32)
        + bl2_ref[...]
        + jnp.dot(h, wr2_ref[...], preferred_element_type=jnp.float32), 0.0)

  return pl.pallas_call(
      body,
      out_shape=jax.ShapeDtypeStruct((_N2P, 64), jnp.float32),
  )(x2p, ei2p, Wl1, bl1, Wr1, Wl2, bl2, Wr2)


def _tc_topk(out1p, out2, batchp):
  def body(o1_ref, o2_ref, b_ref, idx_ref):
    o1 = o1_ref[...]
    o2k = o2_ref[_N2 - 1:_N2, :]
    ones = jnp.ones((1, 64), jnp.float32)
    dotrow = lax.dot_general(o2k, o1, (((1,), (1,)), ((), ())),
                             preferred_element_type=jnp.float32)
    n1row = lax.dot_general(ones, o1 * o1, (((1,), (1,)), ((), ())),
                            preferred_element_type=jnp.float32,
                            precision=lax.Precision.HIGHEST)
    n2k = jnp.sum(o2k * o2k, axis=1, keepdims=True)
    key = jnp.sqrt(jnp.maximum(n1row + n2k - 2.0 * dotrow, 0.0) + 1e-12)
    brow = b_ref[0:1, :]
    iob = lax.broadcasted_iota(jnp.int32, (_B, _N1P), 0)
    inseg = brow == iob
    km = jnp.where(inseg, key, _NEG)
    cnt = jnp.sum(inseg.astype(jnp.int32), axis=1, keepdims=True)
    ipos = lax.broadcasted_iota(jnp.int32, (_B, _N1P), 1)
    cols = []
    for _ in range(_K):
      m = jnp.max(km, axis=1, keepdims=True)
      cand = jnp.where(km == m, ipos, jnp.int32(2**30))
      am = jnp.min(cand, axis=1, keepdims=True)
      cols.append(am)
      km = jnp.where(ipos == am, _NEG, km)
    cols.append(cnt)
    cols.append(jnp.zeros((_B, 64 - _K - 1), jnp.int32))
    idx_ref[...] = jnp.concatenate(cols, axis=1)

  return pl.pallas_call(
      body,
      out_shape=jax.ShapeDtypeStruct((_B, 64), jnp.int32),
  )(out1p, out2, batchp)


def _tc_head(out1p, out2, idxcnt, w1p, b1, g1, be1, w2, b2, g2, be2, w3, b3):
  def ln(x, g, b):
    mu = jnp.mean(x, axis=-1, keepdims=True)
    v = jnp.mean((x - mu) ** 2, axis=-1, keepdims=True)
    return (x - mu) * jax.lax.rsqrt(v + 1e-5) * g + b

  def body(o1_ref, o2_ref, ic_ref, w1_ref, b1_ref, g1_ref, be1_ref,
           w2_ref, b2_ref, g2_ref, be2_ref, w3_ref, b3_ref, o_ref, g_scr):
    for b in range(_B):
      def gather_j(j, _, b=b):
        r = ic_ref[b, j]
        g_scr[pl.ds(j * _B + b, 1), :] = o1_ref[pl.ds(r, 1), :]
        return 0
      lax.fori_loop(0, _K, gather_j, 0)

    g = g_scr[...]
    o2 = o2_ref[...]
    d = lax.dot_general(g, o2, (((1,), (1,)), ((), ())),
                        preferred_element_type=jnp.float32)
    n1 = jnp.sum(g * g, axis=1, keepdims=True)
    n2row = lax.dot_general(jnp.ones((1, 64), jnp.float32), o2 * o2,
                            (((1,), (1,)), ((), ())),
                            preferred_element_type=jnp.float32,
                            precision=lax.Precision.HIGHEST)
    dist = jnp.sqrt(jnp.maximum(n1 + n2row - 2.0 * d, 0.0) + 1e-12)

    ii = lax.broadcasted_iota(jnp.int32, (_K * _B, 1), 0)
    jj = ii // _B
    bb = lax.rem(ii, _B)
    cntcol = jnp.zeros((_K * _B, 1), jnp.int32)
    for b in range(_B):
      cntcol = jnp.where(bb == b, ic_ref[b, _K], cntcol)
    top = jnp.where(jj < cntcol, dist, 0.0)

    y = jnp.zeros((_B, 128), jnp.float32)
    for j in range(_K):
      y = y + jnp.dot(top[j * _B:(j + 1) * _B, :], w1_ref[j],
                      preferred_element_type=jnp.float32)
    y = jnp.maximum(ln(y + b1_ref[...], g1_ref[...], be1_ref[...]), 0.0)
    y = jnp.dot(y, w2_ref[...], preferred_element_type=jnp.float32) + b2_ref[...]
    y = jnp.maximum(ln(y, g2_ref[...], be2_ref[...]), 0.0)
    y = jnp.dot(y, w3_ref[...], preferred_element_type=jnp.float32)[:, 0:1]
    y = y + b3_ref[...]
    o_ref[...] = 1.0 / (1.0 + jnp.exp(-y))

  return pl.pallas_call(
      body,
      in_specs=[
          pl.BlockSpec((_N1P, 64), lambda: (0, 0)),
          pl.BlockSpec((_N2P, 64), lambda: (0, 0)),
          pl.BlockSpec(memory_space=pltpu.SMEM),
          pl.BlockSpec((_K, _N2P, 128), lambda: (0, 0, 0)),
          pl.BlockSpec((1, 128), lambda: (0, 0)),
          pl.BlockSpec((1, 128), lambda: (0, 0)),
          pl.BlockSpec((1, 128), lambda: (0, 0)),
          pl.BlockSpec((128, 64), lambda: (0, 0)),
          pl.BlockSpec((1, 64), lambda: (0, 0)),
          pl.BlockSpec((1, 64), lambda: (0, 0)),
          pl.BlockSpec((1, 64), lambda: (0, 0)),
          pl.BlockSpec((64, 1), lambda: (0, 0)),
          pl.BlockSpec((1, 1), lambda: (0, 0)),
      ],
      out_specs=pl.BlockSpec((_B, 1), lambda: (0, 0)),
      out_shape=jax.ShapeDtypeStruct((_B, 1), jnp.float32),
      scratch_shapes=[pltpu.VMEM((_K * _B, 64), jnp.float32)],
  )(out1p, out2, idxcnt, w1p, b1, g1, be1, w2, b2, g2, be2, w3, b3)


def kernel(x1, edge_index1, batch1, x2, edge_index2, Wl1, bl1, Wr1, Wl2, bl2,
           Wr2, fc1_w, fc1_b, g1, be1, fc2_w, fc2_b, g2, be2, fc3_w, fc3_b):
  src1 = edge_index1[0]
  dst1 = edge_index1[1]
  zf = jnp.zeros((_RRT, _F), jnp.float32)
  onesr = jnp.ones((_CH, _F), jnp.float32)

  cnt1 = _seg_cnt(dst1, zf, onesr).reshape(_NC, _NACC, _F)
  msum1 = _seg_sum(x1, src1, dst1, zf).reshape(_NC, _NACC, _F)

  bl1r = bl1.reshape(1, -1)
  bl2r = bl2.reshape(1, -1)
  h1 = _tc_layer(msum1, cnt1, x1, Wl1, bl1r, Wr1)

  msum2 = _seg_sum(h1, src1, dst1, zf).reshape(_NC, _NACC, _F)
  out1 = _tc_layer(msum2, cnt1, h1, Wl2, bl2r, Wr2)

  x2p = jnp.pad(x2, ((0, _N2P - _N2), (0, 0)))
  ei2p = jnp.pad(edge_index2, ((0, 6), (0, _E2P - edge_index2.shape[1])),
                 constant_values=-1)
  out2 = _tc_graph2(x2p, ei2p, Wl1, bl1r, Wr1, Wl2, bl2r, Wr2)

  out1p = jnp.pad(out1, ((0, _N1P - _N1), (0, 0)))
  batchp = jnp.broadcast_to(
      jnp.pad(batch1, (0, _N1P - _N1), constant_values=_B).reshape(1, _N1P),
      (8, _N1P))
  idxcnt = _tc_topk(out1p, out2, batchp)

  w1p = jnp.pad(fc1_w.reshape(_K, _N2, 128), ((0, 0), (0, _N2P - _N2), (0, 0)))
  return _tc_head(out1p, out2, idxcnt, w1p,
                  fc1_b.reshape(1, -1), g1.reshape(1, -1), be1.reshape(1, -1),
                  fc2_w, fc2_b.reshape(1, -1), g2.reshape(1, -1),
                  be2.reshape(1, -1), fc3_w, fc3_b.reshape(1, -1))


# speedup vs baseline: 4.8397x; 2.1686x over previous
import functools

import jax
import jax.numpy as jnp
from jax import lax
from jax.experimental import pallas as pl
from jax.experimental.pallas import tpu as pltpu
from jax.experimental.pallas import tpu_sc as plsc

_N1 = 10000
_E1 = 640000
_N2 = 199
_F = 128
_B = 16
_K = 50

_N1P = 10240

_NC = 2
_NS = 16
_NW = _NC * _NS
_EPW = _E1 // _NW
_CH = 80
_NCH = _EPW // _CH
_RPT = _N1P // _NS

_N2P = 256
_E2P = 3200

_NEG = float("-inf")


_NR = 3
_RNG = 3584
_NACC = _NR * _RNG
_RRT = _RNG // _NS
_SENT = -1


def _sc_mesh():
  return plsc.VectorSubcoreMesh(
      core_axis_name="c", subcore_axis_name="s",
      num_cores=_NC, num_subcores=_NS)


def _make_seg_kernel():
  scratch = [
      pltpu.VMEM((_EPW,), jnp.int32),
      pltpu.VMEM((_EPW,), jnp.int32),
      pltpu.VMEM((_CH,), jnp.int32),
      pltpu.VMEM((_CH,), jnp.int32),
      pltpu.VMEM((_CH,), jnp.int32),
      pltpu.VMEM((_CH,), jnp.int32),
      pltpu.VMEM((_CH, _F), jnp.float32),
      pltpu.VMEM((_CH, _F), jnp.float32),
      pltpu.SemaphoreType.DMA,
      pltpu.SemaphoreType.DMA,
      pltpu.VMEM((_RRT, _F), jnp.float32),
      pltpu.VMEM_SHARED((_RNG, _F), jnp.float32),
  ]

  def body(x_hbm, src_hbm, dst_hbm, zf_hbm, msum_hbm,
           src_all, dst_all, sadj0, dadj0, sadj1, dadj1, rows0, rows1,
           sem0, sem1, bounce, accf):
    c = lax.axis_index("c")
    s = lax.axis_index("s")
    w = s * _NC + c

    pltpu.sync_copy(src_hbm.at[pl.ds(w * _EPW, _EPW)], src_all)
    pltpu.sync_copy(dst_hbm.at[pl.ds(w * _EPW, _EPW)], dst_all)
    pltpu.sync_copy(zf_hbm, bounce)

    def adj(off, lo, hi, sadj_v, dadj_v):
      @pl.loop(0, _CH // 16)
      def _adj(j):
        sl = pl.ds(off + j * 16, 16)
        dv = dst_all[sl]
        sv = src_all[sl]
        m = (dv >= lo) & (dv < hi)
        osl = pl.ds(j * 16, 16)
        sadj_v[osl] = jnp.where(m, sv, _SENT)
        dadj_v[osl] = jnp.where(m, dv - lo, _SENT)

    for p in range(_NR):
      lo = p * _RNG
      hi = lo + _RNG
      pltpu.sync_copy(bounce, accf.at[pl.ds(s * _RRT, _RRT)])
      plsc.subcore_barrier()

      @pl.loop(0, _NCH // 2)
      def _step(i):
        o0 = (2 * i) * _CH
        o1 = (2 * i + 1) * _CH
        adj(o0, lo, hi, sadj0, dadj0)
        g0 = pltpu.async_copy(
            x_hbm.at[plsc.Indices(sadj0, ignored_value=_SENT)], rows0, sem0)
        adj(o1, lo, hi, sadj1, dadj1)
        g1 = pltpu.async_copy(
            x_hbm.at[plsc.Indices(sadj1, ignored_value=_SENT)], rows1, sem1)
        g0.wait()
        pltpu.sync_copy(
            rows0, accf.at[plsc.Indices(dadj0, ignored_value=_SENT)], add=True)
        g1.wait()
        pltpu.sync_copy(
            rows1, accf.at[plsc.Indices(dadj1, ignored_value=_SENT)], add=True)

      plsc.subcore_barrier()
      pltpu.sync_copy(accf.at[pl.ds(s * _RRT, _RRT)], bounce)
      pltpu.sync_copy(bounce,
                      msum_hbm.at[pl.ds(c * _NACC + lo + s * _RRT, _RRT)])
      plsc.subcore_barrier()
      if p + 1 < _NR:
        pltpu.sync_copy(zf_hbm, bounce)

  return pl.kernel(
      body,
      out_type=jax.ShapeDtypeStruct((_NC * _NACC, _F), jnp.float32),
      mesh=_sc_mesh(),
      scratch_types=scratch,
  )


def _make_cnt_kernel():
  scratch = [
      pltpu.VMEM((_EPW,), jnp.int32),
      pltpu.VMEM((_CH,), jnp.int32),
      pltpu.VMEM((_CH, _F), jnp.float32),
      pltpu.VMEM((_RRT, _F), jnp.float32),
      pltpu.VMEM_SHARED((_RNG, _F), jnp.float32),
  ]

  def body(dst_hbm, zf_hbm, ones_hbm, cnt_hbm,
           dst_all, dadj_v, ones_v, bounce, accc):
    c = lax.axis_index("c")
    s = lax.axis_index("s")
    w = s * _NC + c

    pltpu.sync_copy(dst_hbm.at[pl.ds(w * _EPW, _EPW)], dst_all)
    pltpu.sync_copy(ones_hbm, ones_v)
    pltpu.sync_copy(zf_hbm, bounce)
    for p in range(_NR):
      lo = p * _RNG
      hi = lo + _RNG
      pltpu.sync_copy(bounce, accc.at[pl.ds(s * _RRT, _RRT)])
      plsc.subcore_barrier()

      @pl.loop(0, _NCH)
      def _step(i):
        @pl.loop(0, _CH // 16)
        def _adj(j):
          sl = pl.ds(i * _CH + j * 16, 16)
          dv = dst_all[sl]
          m = (dv >= lo) & (dv < hi)
          dadj_v[pl.ds(j * 16, 16)] = jnp.where(m, dv - lo, _SENT)

        pltpu.sync_copy(
            ones_v,
            accc.at[plsc.Indices(dadj_v, ignored_value=_SENT)], add=True)

      plsc.subcore_barrier()
      pltpu.sync_copy(accc.at[pl.ds(s * _RRT, _RRT)], bounce)
      pltpu.sync_copy(bounce,
                      cnt_hbm.at[pl.ds(c * _NACC + lo + s * _RRT, _RRT)])
      plsc.subcore_barrier()
      if p + 1 < _NR:
        pltpu.sync_copy(zf_hbm, bounce)

  return pl.kernel(
      body,
      out_type=jax.ShapeDtypeStruct((_NC * _NACC, _F), jnp.float32),
      mesh=_sc_mesh(),
      scratch_types=scratch,
  )


_seg_cache = {}


def _seg_sum(*args):
  if "seg" not in _seg_cache:
    _seg_cache["seg"] = _make_seg_kernel()
  return _seg_cache["seg"](*args)


def _seg_cnt(*args):
  if "cnt" not in _seg_cache:
    _seg_cache["cnt"] = _make_cnt_kernel()
  return _seg_cache["cnt"](*args)


def _tc_layer(msum_p, cnt_p, x, Wl, bl, Wr):
  n, f = x.shape
  fo = Wl.shape[1]
  blk = 2000
  grid = n // blk

  def body(ms_ref, cp_ref, x_ref, wl_ref, bl_ref, wr_ref, o_ref):
    ms = ms_ref[0] + ms_ref[1]
    cnt = cp_ref[0][:, 0:1] + cp_ref[1][:, 0:1]
    mean = ms / jnp.maximum(cnt, 1.0)
    h = (jnp.dot(mean, wl_ref[...], preferred_element_type=jnp.float32)
         + bl_ref[...]
         + jnp.dot(x_ref[...], wr_ref[...], preferred_element_type=jnp.float32))
    o_ref[...] = jnp.maximum(h, 0.0)

  return pl.pallas_call(
      body,
      grid=(grid,),
      in_specs=[
          pl.BlockSpec((2, blk, f), lambda i: (0, i, 0)),
          pl.BlockSpec((2, blk, f), lambda i: (0, i, 0)),
          pl.BlockSpec((blk, f), lambda i: (i, 0)),
          pl.BlockSpec((f, fo), lambda i: (0, 0)),
          pl.BlockSpec((1, fo), lambda i: (0, 0)),
          pl.BlockSpec((f, fo), lambda i: (0, 0)),
      ],
      out_specs=pl.BlockSpec((blk, fo), lambda i: (i, 0)),
      out_shape=jax.ShapeDtypeStruct((n, fo), jnp.float32),
  )(msum_p, cnt_p, x, Wl, bl, Wr)


def _tc_graph2(x2p, ei2p, Wl1, bl1, Wr1, Wl2, bl2, Wr2):
  def body(x_ref, ei_ref, wl1_ref, bl1_ref, wr1_ref, wl2_ref, bl2_ref,
           wr2_ref, o_ref):
    src = ei_ref[0:1, :]
    dst = ei_ref[1:2, :]
    ionn = lax.broadcasted_iota(jnp.int32, (_N2P, _E2P), 0)
    dm = (dst == ionn).astype(jnp.float32)
    sm = (src == ionn).astype(jnp.float32)
    a = lax.dot_general(dm, sm, (((1,), (1,)), ((), ())),
                        preferred_element_type=jnp.float32,
                        precision=lax.Precision.HIGHEST)
    cnt2 = jnp.maximum(jnp.sum(a, axis=1, keepdims=True), 1.0)
    x2 = x_ref[...]
    ms1 = jnp.dot(a, x2, preferred_element_type=jnp.float32,
                  precision=lax.Precision.HIGHEST)
    h = jnp.maximum(
        jnp.dot(ms1 / cnt2, wl1_ref[...], preferred_element_type=jnp.float32)
        + bl1_ref[...]
        + jnp.dot(x2, wr1_ref[...], preferred_element_type=jnp.float32), 0.0)
    ms2 = jnp.dot(a, h, preferred_element_type=jnp.float32,
                  precision=lax.Precision.HIGHEST)
    o_ref[...] = jnp.maximum(
        jnp.dot(ms2 / cnt2, wl2_ref[...], preferred_element_type=jnp.float32)
        + bl2_ref[...]
        + jnp.dot(h, wr2_ref[...], preferred_element_type=jnp.float32), 0.0)

  return pl.pallas_call(
      body,
      out_shape=jax.ShapeDtypeStruct((_N2P, 64), jnp.float32),
  )(x2p, ei2p, Wl1, bl1, Wr1, Wl2, bl2, Wr2)


def _tc_topk(out1p, out2, batchp):
  def body(o1_ref, o2_ref, b_ref, idx_ref):
    o1 = o1_ref[...]
    o2k = o2_ref[_N2 - 1:_N2, :]
    ones = jnp.ones((1, 64), jnp.float32)
    dotrow = lax.dot_general(o2k, o1, (((1,), (1,)), ((), ())),
                             preferred_element_type=jnp.float32)
    n1row = lax.dot_general(ones, o1 * o1, (((1,), (1,)), ((), ())),
                            preferred_element_type=jnp.float32,
                            precision=lax.Precision.HIGHEST)
    n2k = jnp.sum(o2k * o2k, axis=1, keepdims=True)
    key = jnp.sqrt(jnp.maximum(n1row + n2k - 2.0 * dotrow, 0.0) + 1e-12)
    brow = b_ref[0:1, :]
    iob = lax.broadcasted_iota(jnp.int32, (_B, _N1P), 0)
    inseg = brow == iob
    km = jnp.where(inseg, key, _NEG)
    cnt = jnp.sum(inseg.astype(jnp.int32), axis=1, keepdims=True)
    ipos = lax.broadcasted_iota(jnp.int32, (_B, _N1P), 1)
    cols = []
    for _ in range(_K):
      m = jnp.max(km, axis=1, keepdims=True)
      cand = jnp.where(km == m, ipos, jnp.int32(2**30))
      am = jnp.min(cand, axis=1, keepdims=True)
      cols.append(am)
      km = jnp.where(ipos == am, _NEG, km)
    cols.append(cnt)
    cols.append(jnp.zeros((_B, 64 - _K - 1), jnp.int32))
    idx_ref[...] = jnp.concatenate(cols, axis=1)

  return pl.pallas_call(
      body,
      out_shape=jax.ShapeDtypeStruct((_B, 64), jnp.int32),
  )(out1p, out2, batchp)


def _tc_head(out1p, out2, idxcnt, w1p, b1, g1, be1, w2, b2, g2, be2, w3, b3):
  def ln(x, g, b):
    mu = jnp.mean(x, axis=-1, keepdims=True)
    v = jnp.mean((x - mu) ** 2, axis=-1, keepdims=True)
    return (x - mu) * jax.lax.rsqrt(v + 1e-5) * g + b

  def body(o1_ref, o2_ref, ic_ref, w1_ref, b1_ref, g1_ref, be1_ref,
           w2_ref, b2_ref, g2_ref, be2_ref, w3_ref, b3_ref, o_ref, g_scr):
    for b in range(_B):
      def gather_j(j, _, b=b):
        r = ic_ref[b, j]
        g_scr[pl.ds(j * _B + b, 1), :] = o1_ref[pl.ds(r, 1), :]
        return 0
      lax.fori_loop(0, _K, gather_j, 0)

    g = g_scr[...]
    o2 = o2_ref[...]
    d = lax.dot_general(g, o2, (((1,), (1,)), ((), ())),
                        preferred_element_type=jnp.float32)
    n1 = jnp.sum(g * g, axis=1, keepdims=True)
    n2row = lax.dot_general(jnp.ones((1, 64), jnp.float32), o2 * o2,
                            (((1,), (1,)), ((), ())),
                            preferred_element_type=jnp.float32,
                            precision=lax.Precision.HIGHEST)
    dist = jnp.sqrt(jnp.maximum(n1 + n2row - 2.0 * d, 0.0) + 1e-12)

    ii = lax.broadcasted_iota(jnp.int32, (_K * _B, 1), 0)
    jj = ii // _B
    bb = lax.rem(ii, _B)
    cntcol = jnp.zeros((_K * _B, 1), jnp.int32)
    for b in range(_B):
      cntcol = jnp.where(bb == b, ic_ref[b, _K], cntcol)
    top = jnp.where(jj < cntcol, dist, 0.0)

    y = jnp.zeros((_B, 128), jnp.float32)
    for j in range(_K):
      y = y + jnp.dot(top[j * _B:(j + 1) * _B, :], w1_ref[j],
                      preferred_element_type=jnp.float32)
    y = jnp.maximum(ln(y + b1_ref[...], g1_ref[...], be1_ref[...]), 0.0)
    y = jnp.dot(y, w2_ref[...], preferred_element_type=jnp.float32) + b2_ref[...]
    y = jnp.maximum(ln(y, g2_ref[...], be2_ref[...]), 0.0)
    y = jnp.dot(y, w3_ref[...], preferred_element_type=jnp.float32)[:, 0:1]
    y = y + b3_ref[...]
    o_ref[...] = 1.0 / (1.0 + jnp.exp(-y))

  return pl.pallas_call(
      body,
      in_specs=[
          pl.BlockSpec((_N1P, 64), lambda: (0, 0)),
          pl.BlockSpec((_N2P, 64), lambda: (0, 0)),
          pl.BlockSpec(memory_space=pltpu.SMEM),
          pl.BlockSpec((_K, _N2P, 128), lambda: (0, 0, 0)),
          pl.BlockSpec((1, 128), lambda: (0, 0)),
          pl.BlockSpec((1, 128), lambda: (0, 0)),
          pl.BlockSpec((1, 128), lambda: (0, 0)),
          pl.BlockSpec((128, 64), lambda: (0, 0)),
          pl.BlockSpec((1, 64), lambda: (0, 0)),
          pl.BlockSpec((1, 64), lambda: (0, 0)),
          pl.BlockSpec((1, 64), lambda: (0, 0)),
          pl.BlockSpec((64, 1), lambda: (0, 0)),
          pl.BlockSpec((1, 1), lambda: (0, 0)),
      ],
      out_specs=pl.BlockSpec((_B, 1), lambda: (0, 0)),
      out_shape=jax.ShapeDtypeStruct((_B, 1), jnp.float32),
      scratch_shapes=[pltpu.VMEM((_K * _B, 64), jnp.float32)],
  )(out1p, out2, idxcnt, w1p, b1, g1, be1, w2, b2, g2, be2, w3, b3)


def kernel(x1, edge_index1, batch1, x2, edge_index2, Wl1, bl1, Wr1, Wl2, bl2,
           Wr2, fc1_w, fc1_b, g1, be1, fc2_w, fc2_b, g2, be2, fc3_w, fc3_b):
  src1 = edge_index1[0]
  dst1 = edge_index1[1]
  zf = jnp.zeros((_RRT, _F), jnp.float32)
  onesr = jnp.ones((_CH, _F), jnp.float32)

  cnt1 = _seg_cnt(dst1, zf, onesr).reshape(_NC, _NACC, _F)
  msum1 = _seg_sum(x1, src1, dst1, zf).reshape(_NC, _NACC, _F)

  bl1r = bl1.reshape(1, -1)
  bl2r = bl2.reshape(1, -1)
  h1 = _tc_layer(msum1, cnt1, x1, Wl1, bl1r, Wr1)

  msum2 = _seg_sum(h1, src1, dst1, zf).reshape(_NC, _NACC, _F)
  out1 = _tc_layer(msum2, cnt1, h1, Wl2, bl2r, Wr2)

  x2p = jnp.pad(x2, ((0, _N2P - _N2), (0, 0)))
  ei2p = jnp.pad(edge_index2, ((0, 6), (0, _E2P - edge_index2.shape[1])),
                 constant_values=-1)
  out2 = _tc_graph2(x2p, ei2p, Wl1, bl1r, Wr1, Wl2, bl2r, Wr2)

  out1p = jnp.pad(out1, ((0, _N1P - _N1), (0, 0)))
  batchp = jnp.broadcast_to(
      jnp.pad(batch1, (0, _N1P - _N1), constant_values=_B).reshape(1, _N1P),
      (8, _N1P))
  idxcnt = _tc_topk(out1p, out2, batchp)

  w1p = jnp.pad(fc1_w.reshape(_K, _N2, 128), ((0, 0), (0, _N2P - _N2), (0, 0)))
  return _tc_head(out1p, out2, idxcnt, w1p,
                  fc1_b.reshape(1, -1), g1.reshape(1, -1), be1.reshape(1, -1),
                  fc2_w, fc2_b.reshape(1, -1), g2.reshape(1, -1),
                  be2.reshape(1, -1), fc3_w, fc3_b.reshape(1, -1))
